# baseline (speedup 1.0000x reference)
import functools

import jax
import jax.numpy as jnp
from jax import lax
from jax.experimental import pallas as pl
from jax.experimental.pallas import tpu as pltpu
from jax.experimental.pallas import tpu_sc as plsc

_NC = 2
_NS = 16
_NW = _NC * _NS
_CHUNK = 128


def kernel(p_h, p_t, p_r, n_h, n_t, n_r, ent_emb, rel_emb):
    B = p_h.shape[0]
    D = ent_emb.shape[1]
    bpw = B // _NW
    n_chunks = bpw // _CHUNK
    groups = bpw // 16

    mesh = plsc.VectorSubcoreMesh(
        core_axis_name="c", subcore_axis_name="s",
        num_cores=_NC, num_subcores=_NS)

    out_t = jax.ShapeDtypeStruct((B,), jnp.float32)
    scratch = (
        [pltpu.VMEM((n_chunks, _CHUNK), jnp.int32) for _ in range(6)]
        + [pltpu.VMEM((bpw, D), jnp.float32) for _ in range(6)]
        + [pltpu.VMEM((bpw,), jnp.float32) for _ in range(2)]
        + [pltpu.VMEM((bpw * 16,), jnp.float32)]
        + [pltpu.SemaphoreType.DMA]
    )

    @functools.partial(
        pl.kernel,
        out_type=(out_t, out_t),
        mesh=mesh,
        scratch_types=scratch,
        compiler_params=pltpu.CompilerParams(
            needs_layout_passes=False, use_tc_tiling_on_sc=False),
    )
    def run(ph_h, pt_h, pr_h, nh_h, nt_h, nr_h, ent_h, rel_h,
            po_h, no_h,
            iv0, iv1, iv2, iv3, iv4, iv5,
            rv0, rv1, rv2, rv3, rv4, rv5,
            op_v, on_v, dred_v, sem):
        wid = lax.axis_index("s") * _NC + lax.axis_index("c")
        base = wid * bpw

        idx_hbm = [ph_h, pt_h, pr_h, nh_h, nt_h, nr_h]
        idx_v = [iv0, iv1, iv2, iv3, iv4, iv5]
        tables = [ent_h, ent_h, rel_h, ent_h, ent_h, rel_h]
        rows_v = [rv0, rv1, rv2, rv3, rv4, rv5]

        for ih, iv in zip(idx_hbm, idx_v):
            for k in range(n_chunks):
                pltpu.sync_copy(ih.at[pl.ds(base + k * _CHUNK, _CHUNK)],
                                iv.at[k])

        copies = []
        for tab, iv, rv in zip(tables, idx_v, rows_v):
            for k in range(n_chunks):
                copies.append(pltpu.async_copy(
                    tab.at[iv.at[k]],
                    rv.at[pl.ds(k * _CHUNK, _CHUNK)],
                    sem))
        for c in copies:
            c.wait()

        iota16 = lax.iota(jnp.int32, 16)

        def score(h_ref, t_ref, r_ref, o_ref):
            def body1(row, carry):
                d = jnp.zeros((16,), jnp.float32)
                for c in range(D // 16):
                    hv = h_ref[row, pl.ds(c * 16, 16)]
                    rr = r_ref[row, pl.ds(c * 16, 16)]
                    tv = t_ref[row, pl.ds(c * 16, 16)]
                    d = d + jnp.abs(hv + rr - tv)
                dred_v[pl.ds(row * 16, 16)] = d
                return carry
            lax.fori_loop(0, bpw, body1, 0)

            def body2(g, carry):
                base_idx = g * 256 + iota16 * 16
                acc = jnp.zeros((16,), jnp.float32)
                for j in range(16):
                    acc = acc + plsc.load_gather(dred_v, [base_idx + j])
                o_ref[pl.ds(g * 16, 16)] = acc
                return carry
            lax.fori_loop(0, groups, body2, 0)

        score(rv0, rv1, rv2, op_v)
        score(rv3, rv4, rv5, on_v)

        pltpu.sync_copy(op_v, po_h.at[pl.ds(base, bpw)])
        pltpu.sync_copy(on_v, no_h.at[pl.ds(base, bpw)])

    return run(p_h, p_t, p_r, n_h, n_t, n_r, ent_emb, rel_emb)
